# Initial kernel scaffold; baseline (speedup 1.0000x reference)
import jax
import jax.numpy as jnp
from jax.experimental import pallas as pl


def kernel(current_time, mgn_h, X_obs, M_obs, i_obs, last_x, last_t, W_gx, W_gh, Wz, Uz, bz, Wr, Ur, br, Wc, Uc, bc):
    raise NotImplementedError("write your pallas kernel here")


# speedup vs baseline: 1.0723x; 1.0723x over previous
import functools

import jax
import jax.numpy as jnp
from jax import lax
from jax.experimental import pallas as pl
from jax.experimental.pallas import tpu as pltpu
from jax.experimental.pallas import tpu_sc as plsc

D = 32
H = 128
B = 16384
N = 100000
NC, NS = 2, 16
NW = NC * NS
BPW = B // NW
KCH = 128
NCH = BPW // KCH
BLK = 2048

_f32 = jnp.float32
_i32 = jnp.int32


def _wid():
    return lax.axis_index("s") * NC + lax.axis_index("c")


@functools.cache
def _sc_kernels():
    mesh = plsc.VectorSubcoreMesh(core_axis_name="c", subcore_axis_name="s",
                                  num_cores=NC, num_subcores=NS)
    gather1 = _make_sc_gather1(mesh)
    gather2 = _make_sc_gather2(mesh)
    scatter = _make_sc_scatter(mesh)
    return gather1, gather2, scatter


def _make_sc_gather1(mesh):
  @functools.partial(
    pl.kernel,
    out_type=(
        jax.ShapeDtypeStruct((B, D), _f32),
        jax.ShapeDtypeStruct((B, D), _f32),
        jax.ShapeDtypeStruct((B, H), _f32),
        jax.ShapeDtypeStruct((B // KCH, KCH), _i32),
    ),
    mesh=mesh,
    compiler_params=pltpu.CompilerParams(use_tc_tiling_on_sc=False),
    scratch_types=[
        pltpu.VMEM((NCH, KCH), _i32),
        pltpu.VMEM((NCH, KCH), _i32),
        pltpu.VMEM((BPW, D), _f32),
        pltpu.VMEM((BPW, D), _f32),
        pltpu.VMEM((BPW, H), _f32),
        pltpu.SemaphoreType.DMA,
    ],
  )
  def sc_gather1(idx_hbm, lt_hbm, lx_hbm, h_hbm, wmem_hbm,
                 lt_o, lx_o, h_o, w_o, idx_v, w_v, lt_v, lx_v, h_v, sem):
    wid = _wid()
    base = wid * BPW
    pltpu.sync_copy(idx_hbm.at[pl.ds(wid * NCH, NCH)], idx_v)
    copies = []
    for j in range(NCH):
        rows = pl.ds(j * KCH, KCH)
        copies.append(pltpu.async_copy(lt_hbm.at[idx_v.at[j]], lt_v.at[rows], sem))
        copies.append(pltpu.async_copy(lx_hbm.at[idx_v.at[j]], lx_v.at[rows], sem))
        copies.append(pltpu.async_copy(h_hbm.at[idx_v.at[j]], h_v.at[rows], sem))
        copies.append(pltpu.async_copy(wmem_hbm.at[idx_v.at[j]], w_v.at[j], sem))
    for c in copies:
        c.wait()
    pltpu.sync_copy(lt_v, lt_o.at[pl.ds(base, BPW)])
    pltpu.sync_copy(lx_v, lx_o.at[pl.ds(base, BPW)])
    pltpu.sync_copy(h_v, h_o.at[pl.ds(base, BPW)])
    pltpu.sync_copy(w_v, w_o.at[pl.ds(wid * NCH, NCH)])

  return sc_gather1


def _make_sc_gather2(mesh):
  @functools.partial(
    pl.kernel,
    out_type=(
        jax.ShapeDtypeStruct((B, D), _f32),
        jax.ShapeDtypeStruct((B, D), _f32),
    ),
    mesh=mesh,
    compiler_params=pltpu.CompilerParams(use_tc_tiling_on_sc=False),
    scratch_types=[
        pltpu.VMEM((NCH, KCH), _i32),
        pltpu.VMEM((BPW, D), _f32),
        pltpu.VMEM((BPW, D), _f32),
        pltpu.SemaphoreType.DMA,
    ],
  )
  def sc_gather2(w_hbm, lxn_hbm, ltn_hbm, lxp_o, ltp_o, w_v, lxp_v, ltp_v, sem):
    wid = _wid()
    base = wid * BPW
    pltpu.sync_copy(w_hbm.at[pl.ds(wid * NCH, NCH)], w_v)
    copies = []
    for j in range(NCH):
        rows = pl.ds(j * KCH, KCH)
        copies.append(pltpu.async_copy(lxn_hbm.at[w_v.at[j]], lxp_v.at[rows], sem))
        copies.append(pltpu.async_copy(ltn_hbm.at[w_v.at[j]], ltp_v.at[rows], sem))
    for c in copies:
        c.wait()
    pltpu.sync_copy(lxp_v, lxp_o.at[pl.ds(base, BPW)])
    pltpu.sync_copy(ltp_v, ltp_o.at[pl.ds(base, BPW)])

  return sc_gather2


def _make_sc_scatter(mesh):
  @functools.partial(
    pl.kernel,
    out_type=(),
    mesh=mesh,
    compiler_params=pltpu.CompilerParams(use_tc_tiling_on_sc=False),
    scratch_types=[
        pltpu.VMEM((NCH, KCH), _i32),
        pltpu.VMEM((NCH, KCH), _i32),
        pltpu.VMEM((BPW, H), _f32),
        pltpu.VMEM((BPW, D), _f32),
        pltpu.VMEM((BPW, D), _f32),
        pltpu.SemaphoreType.DMA,
    ],
  )
  def sc_scatter(idx_hbm, w_hbm, hupd_hbm, lxp_hbm, ltp_hbm,
                 h_ref, lx_ref, lt_ref, idx_v, w_v, hp_v, lxp_v, ltp_v, sem):
    wid = _wid()
    base = wid * BPW
    pltpu.sync_copy(idx_hbm.at[pl.ds(wid * NCH, NCH)], idx_v)
    pltpu.sync_copy(w_hbm.at[pl.ds(wid * NCH, NCH)], w_v)
    pltpu.sync_copy(lxp_hbm.at[pl.ds(base, BPW)], lxp_v)
    pltpu.sync_copy(ltp_hbm.at[pl.ds(base, BPW)], ltp_v)
    gathers = []
    for j in range(NCH):
        rows = pl.ds(j * KCH, KCH)
        gathers.append(pltpu.async_copy(hupd_hbm.at[w_v.at[j]], hp_v.at[rows], sem))
    for c in gathers:
        c.wait()
    scatters = []
    for j in range(NCH):
        rows = pl.ds(j * KCH, KCH)
        scatters.append(pltpu.async_copy(hp_v.at[rows], h_ref.at[idx_v.at[j]], sem))
        scatters.append(pltpu.async_copy(lxp_v.at[rows], lx_ref.at[idx_v.at[j]], sem))
        scatters.append(pltpu.async_copy(ltp_v.at[rows], lt_ref.at[idx_v.at[j]], sem))
    for c in scatters:
        c.wait()

  return sc_scatter


def _tc_a_body(ct_ref, lt_ref, lx_ref, x_ref, m_ref, wgx_ref, wgh_ref,
               lxn_o, ltn_o, gx_o, gh_o, sx_o, sm_o):
    i = pl.program_id(0)
    ct = ct_ref[0, 0]
    lt = lt_ref[...]
    m = m_ref[...]
    x = x_ref[...]
    one_m = 1.0 - m
    lxn_o[...] = lx_ref[...] * one_m + x * m
    ltn_o[...] = lt * one_m + ct * m
    itv = ct - lt
    gx_o[...] = jnp.exp(-jnp.maximum(
        jnp.dot(itv, wgx_ref[...], preferred_element_type=_f32,
                precision=lax.Precision.HIGHEST), 0.0))
    gh_o[...] = jnp.exp(-jnp.maximum(
        jnp.dot(itv, wgh_ref[...], preferred_element_type=_f32,
                precision=lax.Precision.HIGHEST), 0.0))
    psx = jnp.sum(x, axis=0, keepdims=True)
    psm = jnp.sum(m, axis=0, keepdims=True)

    @pl.when(i == 0)
    def _():
        sx_o[...] = psx
        sm_o[...] = psm

    @pl.when(i != 0)
    def _():
        sx_o[...] += psx
        sm_o[...] += psm


_TC_A_KW = dict(
    grid=(B // BLK,),
    in_specs=[
        pl.BlockSpec((1, 1), lambda i: (0, 0)),
        pl.BlockSpec((BLK, D), lambda i: (i, 0)),
        pl.BlockSpec((BLK, D), lambda i: (i, 0)),
        pl.BlockSpec((BLK, D), lambda i: (i, 0)),
        pl.BlockSpec((BLK, D), lambda i: (i, 0)),
        pl.BlockSpec((D, D), lambda i: (0, 0)),
        pl.BlockSpec((D, H), lambda i: (0, 0)),
    ],
    out_specs=[
        pl.BlockSpec((BLK, D), lambda i: (i, 0)),
        pl.BlockSpec((BLK, D), lambda i: (i, 0)),
        pl.BlockSpec((BLK, D), lambda i: (i, 0)),
        pl.BlockSpec((BLK, H), lambda i: (i, 0)),
        pl.BlockSpec((1, D), lambda i: (0, 0)),
        pl.BlockSpec((1, D), lambda i: (0, 0)),
    ],
    out_shape=[
        jax.ShapeDtypeStruct((B, D), _f32),
        jax.ShapeDtypeStruct((B, D), _f32),
        jax.ShapeDtypeStruct((B, D), _f32),
        jax.ShapeDtypeStruct((B, H), _f32),
        jax.ShapeDtypeStruct((1, D), _f32),
        jax.ShapeDtypeStruct((1, D), _f32),
    ],
)
_tc_a = pl.pallas_call(_tc_a_body, **_TC_A_KW)


def _tc_b_body(x_ref, m_ref, gx_ref, gh_ref, h0_ref, lxp_ref, sx_ref, sm_ref,
               wzx_ref, wzm_ref, uz_ref, bz_ref,
               wrx_ref, wrm_ref, ur_ref, br_ref,
               wcx_ref, wcm_ref, uc_ref, bc_ref, out_ref):
    m = m_ref[...]
    x = x_ref[...]
    gx = gx_ref[...]
    one_m = 1.0 - m
    mean_x = sx_ref[...] / (sm_ref[...] + B * 1e-6)
    x_hat = m * x + one_m * gx * lxp_ref[...] + one_m * (1.0 - gx) * mean_x
    h = gh_ref[...] * h0_ref[...]

    def mm(a, w):
        return jnp.dot(a, w[...], preferred_element_type=_f32,
                       precision=lax.Precision.HIGHEST)

    z = jax.nn.sigmoid(mm(x_hat, wzx_ref) + mm(m, wzm_ref) + mm(h, uz_ref)
                       + bz_ref[...])
    r = jax.nn.sigmoid(mm(x_hat, wrx_ref) + mm(m, wrm_ref) + mm(h, ur_ref)
                       + br_ref[...])
    h_tilde = jnp.tanh(mm(x_hat, wcx_ref) + mm(m, wcm_ref) + mm(r * h, uc_ref)
                       + bc_ref[...])
    out_ref[...] = (1.0 - z) * h + z * h_tilde


_TC_B_KW = dict(
    grid=(B // BLK,),
    in_specs=[
        pl.BlockSpec((BLK, D), lambda i: (i, 0)),
        pl.BlockSpec((BLK, D), lambda i: (i, 0)),
        pl.BlockSpec((BLK, D), lambda i: (i, 0)),
        pl.BlockSpec((BLK, H), lambda i: (i, 0)),
        pl.BlockSpec((BLK, H), lambda i: (i, 0)),
        pl.BlockSpec((BLK, D), lambda i: (i, 0)),
        pl.BlockSpec((1, D), lambda i: (0, 0)),
        pl.BlockSpec((1, D), lambda i: (0, 0)),
    ] + [
        spec
        for _ in range(3)
        for spec in (
            pl.BlockSpec((D, H), lambda i: (0, 0)),
            pl.BlockSpec((D, H), lambda i: (0, 0)),
            pl.BlockSpec((H, H), lambda i: (0, 0)),
            pl.BlockSpec((1, H), lambda i: (0, 0)),
        )
    ],
    out_specs=pl.BlockSpec((BLK, H), lambda i: (i, 0)),
    out_shape=jax.ShapeDtypeStruct((B, H), _f32),
)
_tc_b = pl.pallas_call(_tc_b_body, **_TC_B_KW)


def kernel(current_time, mgn_h, X_obs, M_obs, i_obs, last_x, last_t,
           W_gx, W_gh, Wz, Uz, bz, Wr, Ur, br, Wc, Uc, bc):
    idx2d = i_obs.reshape(B // KCH, KCH)
    w_mem = jnp.zeros((N,), _i32).at[i_obs].set(jnp.arange(B, dtype=_i32))

    sc_gather1, sc_gather2, sc_scatter = _sc_kernels()
    lt_g, lx_g, h0_g, w2d = sc_gather1(idx2d, last_t, last_x, mgn_h, w_mem)
    lxn, ltn, gx, gh, sx, sm = _tc_a(
        current_time.reshape(1, 1), lt_g, lx_g, X_obs, M_obs, W_gx, W_gh)
    lxp, ltp = sc_gather2(w2d, lxn, ltn)
    h_upd = _tc_b(
        X_obs, M_obs, gx, gh, h0_g, lxp, sx, sm,
        Wz[:D], Wz[D:], Uz, bz.reshape(1, H),
        Wr[:D], Wr[D:], Ur, br.reshape(1, H),
        Wc[:D], Wc[D:], Uc, bc.reshape(1, H))

    h_ref = jax.new_ref(mgn_h)
    lx_ref = jax.new_ref(last_x)
    lt_ref = jax.new_ref(last_t)
    sc_scatter(idx2d, w2d, h_upd, lxp, ltp, h_ref, lx_ref, lt_ref)
    return h_ref[...], lx_ref[...], lt_ref[...]


# speedup vs baseline: 1.1542x; 1.0764x over previous
import functools

import jax
import jax.numpy as jnp
from jax import lax
from jax.experimental import pallas as pl
from jax.experimental.pallas import tpu as pltpu
from jax.experimental.pallas import tpu_sc as plsc

D = 32
H = 128
B = 16384
N = 100000
NC, NS = 2, 16
NW = NC * NS
BPW = B // NW
KCH = 128
NCH = BPW // KCH
BLK = 2048

_f32 = jnp.float32
_i32 = jnp.int32
_SC_PARAMS = pltpu.CompilerParams(use_tc_tiling_on_sc=False)


def _wid():
    return lax.axis_index("s") * NC + lax.axis_index("c")


@functools.cache
def _sc_kernels():
    mesh = plsc.VectorSubcoreMesh(core_axis_name="c", subcore_axis_name="s",
                                  num_cores=NC, num_subcores=NS)
    return (_make_sc_gather1(mesh), _make_sc_gather2(mesh),
            _make_sc_scatter(mesh))


def _make_sc_gather1(mesh):
  @functools.partial(
    pl.kernel,
    out_type=(
        jax.ShapeDtypeStruct((B, H), _f32),
        jax.ShapeDtypeStruct((B, H), _f32),
        jax.ShapeDtypeStruct((B // KCH, KCH), _i32),
    ),
    mesh=mesh,
    compiler_params=_SC_PARAMS,
    scratch_types=[
        pltpu.VMEM((NCH, KCH), _i32),
        pltpu.VMEM((NCH, KCH), _i32),
        pltpu.VMEM((BPW, D), _f32),
        pltpu.VMEM((BPW, D), _f32),
        pltpu.VMEM((BPW, H), _f32),
        pltpu.SemaphoreType.DMA,
    ],
  )
  def sc_gather1(idx_hbm, wmem_hbm, lt_ref, lx_ref, h_ref,
                 pk_o, h_o, w_o, idx_v, w_v, lt_v, lx_v, h_v, sem):
    wid = _wid()
    base = wid * BPW
    pltpu.sync_copy(idx_hbm.at[pl.ds(wid * NCH, NCH)], idx_v)
    copies = []
    for j in range(NCH):
        rows = pl.ds(j * KCH, KCH)
        copies.append(pltpu.async_copy(lt_ref.at[idx_v.at[j]], lt_v.at[rows], sem))
        copies.append(pltpu.async_copy(lx_ref.at[idx_v.at[j]], lx_v.at[rows], sem))
        copies.append(pltpu.async_copy(h_ref.at[idx_v.at[j]], h_v.at[rows], sem))
        copies.append(pltpu.async_copy(wmem_hbm.at[idx_v.at[j]], w_v.at[j], sem))
    for c in copies:
        c.wait()
    obs = pl.ds(base, BPW)
    pltpu.sync_copy(lt_v, pk_o.at[obs, pl.ds(0, D)])
    pltpu.sync_copy(lx_v, pk_o.at[obs, pl.ds(D, D)])
    pltpu.sync_copy(h_v, h_o.at[obs])
    pltpu.sync_copy(w_v, w_o.at[pl.ds(wid * NCH, NCH)])

  return sc_gather1


def _make_sc_gather2(mesh):
  @functools.partial(
    pl.kernel,
    out_type=jax.ShapeDtypeStruct((B, H), _f32),
    mesh=mesh,
    compiler_params=_SC_PARAMS,
    scratch_types=[
        pltpu.VMEM((NCH, KCH), _i32),
        pltpu.VMEM((BPW, H), _f32),
        pltpu.SemaphoreType.DMA,
    ],
  )
  def sc_gather2(w_hbm, pk2_hbm, pkw_o, w_v, pkw_v, sem):
    wid = _wid()
    base = wid * BPW
    pltpu.sync_copy(w_hbm.at[pl.ds(wid * NCH, NCH)], w_v)
    copies = []
    for j in range(NCH):
        rows = pl.ds(j * KCH, KCH)
        copies.append(pltpu.async_copy(pk2_hbm.at[w_v.at[j]], pkw_v.at[rows], sem))
    for c in copies:
        c.wait()
    pltpu.sync_copy(pkw_v, pkw_o.at[pl.ds(base, BPW)])

  return sc_gather2


def _make_sc_scatter(mesh):
  @functools.partial(
    pl.kernel,
    out_type=(),
    mesh=mesh,
    compiler_params=_SC_PARAMS,
    scratch_types=[
        pltpu.VMEM((NCH, KCH), _i32),
        pltpu.VMEM((NCH, KCH), _i32),
        pltpu.VMEM((BPW, H), _f32),
        pltpu.VMEM((BPW, D), _f32),
        pltpu.VMEM((BPW, D), _f32),
        pltpu.SemaphoreType.DMA,
    ],
  )
  def sc_scatter(idx_hbm, w_hbm, hupd_hbm, pkw_hbm,
                 h_ref, lx_ref, lt_ref,
                 idx_v, w_v, hp_v, lt_s, lx_s, sem):
    wid = _wid()
    base = wid * BPW
    pltpu.sync_copy(idx_hbm.at[pl.ds(wid * NCH, NCH)], idx_v)
    pltpu.sync_copy(w_hbm.at[pl.ds(wid * NCH, NCH)], w_v)
    gathers = []
    for j in range(NCH):
        rows = pl.ds(j * KCH, KCH)
        gathers.append(pltpu.async_copy(hupd_hbm.at[w_v.at[j]], hp_v.at[rows], sem))
    obs = pl.ds(base, BPW)
    pltpu.sync_copy(pkw_hbm.at[obs, pl.ds(0, D)], lt_s)
    pltpu.sync_copy(pkw_hbm.at[obs, pl.ds(D, D)], lx_s)
    for c in gathers:
        c.wait()
    scatters = []
    for j in range(NCH):
        rows = pl.ds(j * KCH, KCH)
        scatters.append(pltpu.async_copy(hp_v.at[rows], h_ref.at[idx_v.at[j]], sem))
        scatters.append(pltpu.async_copy(lx_s.at[rows], lx_ref.at[idx_v.at[j]], sem))
        scatters.append(pltpu.async_copy(lt_s.at[rows], lt_ref.at[idx_v.at[j]], sem))
    for c in scatters:
        c.wait()

  return sc_scatter


def _tc_a_body(ct_ref, pk_ref, x_ref, m_ref, wgx_ref, wgh_ref,
               pk2_o, gh_o, sx_o, sm_o):
    i = pl.program_id(0)
    ct = ct_ref[0, 0]
    pk = pk_ref[...]
    lt = pk[:, :D]
    lx = pk[:, D:2 * D]
    m = m_ref[...]
    x = x_ref[...]
    one_m = 1.0 - m
    lxn = lx * one_m + x * m
    ltn = lt * one_m + ct * m
    itv = ct - lt
    gx = jnp.exp(-jnp.maximum(
        jnp.dot(itv, wgx_ref[...], preferred_element_type=_f32,
                precision=lax.Precision.HIGHEST), 0.0))
    gh_o[...] = jnp.exp(-jnp.maximum(
        jnp.dot(itv, wgh_ref[...], preferred_element_type=_f32,
                precision=lax.Precision.HIGHEST), 0.0))
    pk2_o[...] = jnp.concatenate([ltn, lxn, gx, gx], axis=1)
    psx = jnp.sum(x, axis=0, keepdims=True)
    psm = jnp.sum(m, axis=0, keepdims=True)

    @pl.when(i == 0)
    def _():
        sx_o[...] = psx
        sm_o[...] = psm

    @pl.when(i != 0)
    def _():
        sx_o[...] += psx
        sm_o[...] += psm


_TC_A_KW = dict(
    grid=(B // BLK,),
    in_specs=[
        pl.BlockSpec((1, 1), lambda i: (0, 0)),
        pl.BlockSpec((BLK, H), lambda i: (i, 0)),
        pl.BlockSpec((BLK, D), lambda i: (i, 0)),
        pl.BlockSpec((BLK, D), lambda i: (i, 0)),
        pl.BlockSpec((D, D), lambda i: (0, 0)),
        pl.BlockSpec((D, H), lambda i: (0, 0)),
    ],
    out_specs=[
        pl.BlockSpec((BLK, H), lambda i: (i, 0)),
        pl.BlockSpec((BLK, H), lambda i: (i, 0)),
        pl.BlockSpec((1, D), lambda i: (0, 0)),
        pl.BlockSpec((1, D), lambda i: (0, 0)),
    ],
    out_shape=[
        jax.ShapeDtypeStruct((B, H), _f32),
        jax.ShapeDtypeStruct((B, H), _f32),
        jax.ShapeDtypeStruct((1, D), _f32),
        jax.ShapeDtypeStruct((1, D), _f32),
    ],
)
_tc_a = pl.pallas_call(_tc_a_body, **_TC_A_KW)


def _tc_b_body(x_ref, m_ref, gh_ref, h0_ref, pkw_ref, sx_ref, sm_ref,
               wzx_ref, wzm_ref, uz_ref, bz_ref,
               wrx_ref, wrm_ref, ur_ref, br_ref,
               wcx_ref, wcm_ref, uc_ref, bc_ref, out_ref):
    m = m_ref[...]
    x = x_ref[...]
    pkw = pkw_ref[...]
    lxp = pkw[:, D:2 * D]
    gx = pkw[:, 2 * D:3 * D]
    one_m = 1.0 - m
    mean_x = sx_ref[...] / (sm_ref[...] + B * 1e-6)
    x_hat = m * x + one_m * gx * lxp + one_m * (1.0 - gx) * mean_x
    h = gh_ref[...] * h0_ref[...]

    def mm(a, w):
        return jnp.dot(a, w[...], preferred_element_type=_f32,
                       precision=lax.Precision.HIGHEST)

    z = jax.nn.sigmoid(mm(x_hat, wzx_ref) + mm(m, wzm_ref) + mm(h, uz_ref)
                       + bz_ref[...])
    r = jax.nn.sigmoid(mm(x_hat, wrx_ref) + mm(m, wrm_ref) + mm(h, ur_ref)
                       + br_ref[...])
    h_tilde = jnp.tanh(mm(x_hat, wcx_ref) + mm(m, wcm_ref) + mm(r * h, uc_ref)
                       + bc_ref[...])
    out_ref[...] = (1.0 - z) * h + z * h_tilde


_TC_B_KW = dict(
    grid=(B // BLK,),
    in_specs=[
        pl.BlockSpec((BLK, D), lambda i: (i, 0)),
        pl.BlockSpec((BLK, D), lambda i: (i, 0)),
        pl.BlockSpec((BLK, H), lambda i: (i, 0)),
        pl.BlockSpec((BLK, H), lambda i: (i, 0)),
        pl.BlockSpec((BLK, H), lambda i: (i, 0)),
        pl.BlockSpec((1, D), lambda i: (0, 0)),
        pl.BlockSpec((1, D), lambda i: (0, 0)),
    ] + [
        spec
        for _ in range(3)
        for spec in (
            pl.BlockSpec((D, H), lambda i: (0, 0)),
            pl.BlockSpec((D, H), lambda i: (0, 0)),
            pl.BlockSpec((H, H), lambda i: (0, 0)),
            pl.BlockSpec((1, H), lambda i: (0, 0)),
        )
    ],
    out_specs=pl.BlockSpec((BLK, H), lambda i: (i, 0)),
    out_shape=jax.ShapeDtypeStruct((B, H), _f32),
)
_tc_b = pl.pallas_call(_tc_b_body, **_TC_B_KW)


def kernel(current_time, mgn_h, X_obs, M_obs, i_obs, last_x, last_t,
           W_gx, W_gh, Wz, Uz, bz, Wr, Ur, br, Wc, Uc, bc):
    idx2d = i_obs.reshape(B // KCH, KCH)
    w_mem = jnp.zeros((N,), _i32).at[i_obs].max(jnp.arange(B, dtype=_i32))

    h_ref = jax.new_ref(mgn_h)
    lx_ref = jax.new_ref(last_x)
    lt_ref = jax.new_ref(last_t)

    sc_gather1, sc_gather2, sc_scatter = _sc_kernels()
    pk1, h0_g, w2d = sc_gather1(idx2d, w_mem, lt_ref, lx_ref, h_ref)
    pk2, gh, sx, sm = _tc_a(
        current_time.reshape(1, 1), pk1, X_obs, M_obs, W_gx, W_gh)
    pkw = sc_gather2(w2d, pk2)
    h_upd = _tc_b(
        X_obs, M_obs, gh, h0_g, pkw, sx, sm,
        Wz[:D], Wz[D:], Uz, bz.reshape(1, H),
        Wr[:D], Wr[D:], Ur, br.reshape(1, H),
        Wc[:D], Wc[D:], Uc, bc.reshape(1, H))
    sc_scatter(idx2d, w2d, h_upd, pkw, h_ref, lx_ref, lt_ref)
    return h_ref[...], lx_ref[...], lt_ref[...]


# speedup vs baseline: 1.2635x; 1.0947x over previous
import functools

import jax
import jax.numpy as jnp
from jax import lax
from jax.experimental import pallas as pl
from jax.experimental.pallas import tpu as pltpu
from jax.experimental.pallas import tpu_sc as plsc

D = 32
H = 128
B = 16384
N = 100000
NC, NS = 2, 16
NW = NC * NS
BPW = B // NW
KCH = 128
NCH = BPW // KCH
BLK = 2048

_f32 = jnp.float32
_i32 = jnp.int32
_SC_PARAMS = pltpu.CompilerParams(use_tc_tiling_on_sc=False)


def _wid():
    return lax.axis_index("s") * NC + lax.axis_index("c")


@functools.cache
def _sc_kernels():
    mesh = plsc.VectorSubcoreMesh(core_axis_name="c", subcore_axis_name="s",
                                  num_cores=NC, num_subcores=NS)
    return _make_sc_gather(mesh), _make_sc_scatter(mesh)


def _make_sc_gather(mesh):
  @functools.partial(
    pl.kernel,
    out_type=(
        jax.ShapeDtypeStruct((B, H), _f32),
        jax.ShapeDtypeStruct((B, H), _f32),
    ),
    mesh=mesh,
    compiler_params=_SC_PARAMS,
    scratch_types=[
        pltpu.VMEM((NCH, KCH), _i32),
        pltpu.VMEM((BPW, D), _f32),
        pltpu.VMEM((BPW, D), _f32),
        pltpu.VMEM((BPW, H), _f32),
        pltpu.SemaphoreType.DMA,
    ],
  )
  def sc_gather(idx_hbm, lt_ref, lx_ref, h_ref,
                pk_o, h_o, idx_v, lt_v, lx_v, h_v, sem):
    wid = _wid()
    base = wid * BPW
    pltpu.sync_copy(idx_hbm.at[pl.ds(wid * NCH, NCH)], idx_v)
    copies = []
    for j in range(NCH):
        rows = pl.ds(j * KCH, KCH)
        copies.append(pltpu.async_copy(lt_ref.at[idx_v.at[j]], lt_v.at[rows], sem))
        copies.append(pltpu.async_copy(lx_ref.at[idx_v.at[j]], lx_v.at[rows], sem))
        copies.append(pltpu.async_copy(h_ref.at[idx_v.at[j]], h_v.at[rows], sem))
    for c in copies:
        c.wait()
    obs = pl.ds(base, BPW)
    pltpu.sync_copy(lt_v, pk_o.at[obs, pl.ds(0, D)])
    pltpu.sync_copy(lx_v, pk_o.at[obs, pl.ds(D, D)])
    pltpu.sync_copy(h_v, h_o.at[obs])

  return sc_gather


def _make_sc_scatter(mesh):
  @functools.partial(
    pl.kernel,
    out_type=(),
    mesh=mesh,
    compiler_params=_SC_PARAMS,
    scratch_types=[
        pltpu.VMEM((NCH, KCH), _i32),
        pltpu.VMEM((NCH, KCH), _i32),
        pltpu.VMEM((BPW, H), _f32),
        pltpu.VMEM((BPW, D), _f32),
        pltpu.VMEM((BPW, D), _f32),
        pltpu.SemaphoreType.DMA,
    ],
  )
  def sc_scatter(idx_hbm, wmem_hbm, pk2_hbm, hupd_hbm,
                 h_ref, lx_ref, lt_ref,
                 idx_v, w_v, buf_v, lt_s, lx_s, sem):
    wid = _wid()
    base = wid * BPW
    pltpu.sync_copy(idx_hbm.at[pl.ds(wid * NCH, NCH)], idx_v)
    wcopies = [
        pltpu.async_copy(wmem_hbm.at[idx_v.at[j]], w_v.at[j], sem)
        for j in range(NCH)
    ]
    for c in wcopies:
        c.wait()
    pkcopies = [
        pltpu.async_copy(pk2_hbm.at[w_v.at[j]],
                         buf_v.at[pl.ds(j * KCH, KCH)], sem)
        for j in range(NCH)
    ]
    for c in pkcopies:
        c.wait()

    @pl.loop(0, BPW)
    def _(r):
        for c in range(D // 16):
            lt_s[r, pl.ds(c * 16, 16)] = buf_v[r, pl.ds(c * 16, 16)]
            lx_s[r, pl.ds(c * 16, 16)] = buf_v[r, pl.ds(D + c * 16, 16)]

    hcopies = [
        pltpu.async_copy(hupd_hbm.at[w_v.at[j]],
                         buf_v.at[pl.ds(j * KCH, KCH)], sem)
        for j in range(NCH)
    ]
    for c in hcopies:
        c.wait()
    scatters = []
    for j in range(NCH):
        rows = pl.ds(j * KCH, KCH)
        scatters.append(pltpu.async_copy(buf_v.at[rows], h_ref.at[idx_v.at[j]], sem))
        scatters.append(pltpu.async_copy(lx_s.at[rows], lx_ref.at[idx_v.at[j]], sem))
        scatters.append(pltpu.async_copy(lt_s.at[rows], lt_ref.at[idx_v.at[j]], sem))
    for c in scatters:
        c.wait()

  return sc_scatter


def _tc_stats_body(x_ref, m_ref, sx_o, sm_o):
    i = pl.program_id(0)
    psx = jnp.sum(x_ref[...], axis=0, keepdims=True)
    psm = jnp.sum(m_ref[...], axis=0, keepdims=True)

    @pl.when(i == 0)
    def _():
        sx_o[...] = psx
        sm_o[...] = psm

    @pl.when(i != 0)
    def _():
        sx_o[...] += psx
        sm_o[...] += psm


_TC_STATS_KW = dict(
    grid=(B // BLK,),
    in_specs=[
        pl.BlockSpec((BLK, D), lambda i: (i, 0)),
        pl.BlockSpec((BLK, D), lambda i: (i, 0)),
    ],
    out_specs=[
        pl.BlockSpec((1, D), lambda i: (0, 0)),
        pl.BlockSpec((1, D), lambda i: (0, 0)),
    ],
    out_shape=[
        jax.ShapeDtypeStruct((1, D), _f32),
        jax.ShapeDtypeStruct((1, D), _f32),
    ],
)
_tc_stats = pl.pallas_call(_tc_stats_body, **_TC_STATS_KW)


def _tc_main_body(ct_ref, pk_ref, x_ref, m_ref, h0_ref, sx_ref, sm_ref,
                  wgx_ref, wgh_ref,
                  wzx_ref, wzm_ref, uz_ref, bz_ref,
                  wrx_ref, wrm_ref, ur_ref, br_ref,
                  wcx_ref, wcm_ref, uc_ref, bc_ref,
                  pk2_o, hupd_o):
    ct = ct_ref[0, 0]
    pk = pk_ref[...]
    lt = pk[:, :D]
    lx = pk[:, D:2 * D]
    m = m_ref[...]
    x = x_ref[...]
    one_m = 1.0 - m
    lxn = lx * one_m + x * m
    ltn = lt * one_m + ct * m
    itv = ct - lt

    def mm(a, w):
        return jnp.dot(a, w[...], preferred_element_type=_f32,
                       precision=lax.Precision.HIGHEST)

    gx = jnp.exp(-jnp.maximum(mm(itv, wgx_ref), 0.0))
    gh = jnp.exp(-jnp.maximum(mm(itv, wgh_ref), 0.0))
    pk2_o[...] = jnp.concatenate([ltn, lxn, gx, gx], axis=1)
    mean_x = sx_ref[...] / (sm_ref[...] + B * 1e-6)
    x_hat = m * x + one_m * gx * lxn + one_m * (1.0 - gx) * mean_x
    h = gh * h0_ref[...]
    z = jax.nn.sigmoid(mm(x_hat, wzx_ref) + mm(m, wzm_ref) + mm(h, uz_ref)
                       + bz_ref[...])
    r = jax.nn.sigmoid(mm(x_hat, wrx_ref) + mm(m, wrm_ref) + mm(h, ur_ref)
                       + br_ref[...])
    h_tilde = jnp.tanh(mm(x_hat, wcx_ref) + mm(m, wcm_ref) + mm(r * h, uc_ref)
                       + bc_ref[...])
    hupd_o[...] = (1.0 - z) * h + z * h_tilde


_TC_MAIN_KW = dict(
    grid=(B // BLK,),
    in_specs=[
        pl.BlockSpec((1, 1), lambda i: (0, 0)),
        pl.BlockSpec((BLK, H), lambda i: (i, 0)),
        pl.BlockSpec((BLK, D), lambda i: (i, 0)),
        pl.BlockSpec((BLK, D), lambda i: (i, 0)),
        pl.BlockSpec((BLK, H), lambda i: (i, 0)),
        pl.BlockSpec((1, D), lambda i: (0, 0)),
        pl.BlockSpec((1, D), lambda i: (0, 0)),
        pl.BlockSpec((D, D), lambda i: (0, 0)),
        pl.BlockSpec((D, H), lambda i: (0, 0)),
    ] + [
        spec
        for _ in range(3)
        for spec in (
            pl.BlockSpec((D, H), lambda i: (0, 0)),
            pl.BlockSpec((D, H), lambda i: (0, 0)),
            pl.BlockSpec((H, H), lambda i: (0, 0)),
            pl.BlockSpec((1, H), lambda i: (0, 0)),
        )
    ],
    out_specs=[
        pl.BlockSpec((BLK, H), lambda i: (i, 0)),
        pl.BlockSpec((BLK, H), lambda i: (i, 0)),
    ],
    out_shape=[
        jax.ShapeDtypeStruct((B, H), _f32),
        jax.ShapeDtypeStruct((B, H), _f32),
    ],
)
_tc_main = pl.pallas_call(_tc_main_body, **_TC_MAIN_KW)


def kernel(current_time, mgn_h, X_obs, M_obs, i_obs, last_x, last_t,
           W_gx, W_gh, Wz, Uz, bz, Wr, Ur, br, Wc, Uc, bc):
    idx2d = i_obs.reshape(B // KCH, KCH)
    w_mem = jnp.zeros((N,), _i32).at[i_obs].max(jnp.arange(B, dtype=_i32))

    h_ref = jax.new_ref(mgn_h)
    lx_ref = jax.new_ref(last_x)
    lt_ref = jax.new_ref(last_t)

    sc_gather, sc_scatter = _sc_kernels()
    pk1, h0_g = sc_gather(idx2d, lt_ref, lx_ref, h_ref)
    sx, sm = _tc_stats(X_obs, M_obs)
    pk2, h_upd = _tc_main(
        current_time.reshape(1, 1), pk1, X_obs, M_obs, h0_g, sx, sm,
        W_gx, W_gh,
        Wz[:D], Wz[D:], Uz, bz.reshape(1, H),
        Wr[:D], Wr[D:], Ur, br.reshape(1, H),
        Wc[:D], Wc[D:], Uc, bc.reshape(1, H))
    sc_scatter(idx2d, w_mem, pk2, h_upd, h_ref, lx_ref, lt_ref)
    return h_ref[...], lx_ref[...], lt_ref[...]
